# baseline (speedup 1.0000x reference)
import functools

import jax
import jax.numpy as jnp
from jax import lax
from jax.experimental import pallas as pl
from jax.experimental.pallas import tpu as pltpu
from jax.experimental.pallas import tpu_sc as plsc

_D = 64
_WIDE = 128
_CHUNK = 128
_LANES = 16
_NGRP = _CHUNK // _LANES
_NIN = 4
_NOUT = 2
_EPS = 1e-5


def _rsqrt(x):
    i = plsc.bitcast(x, jnp.int32)
    i = jnp.int32(0x5F3759DF) - lax.shift_right_logical(i, 1)
    y = plsc.bitcast(i, jnp.float32)
    half = x * 0.5
    for _ in range(3):
        y = y * (1.5 - half * y * y)
    return y


def kernel(x, emb_table, ln_weight, ln_bias):
    batch, seq = x.shape
    n_rows = batch * seq
    n_emb = emb_table.shape[0]
    info = plsc.get_sparse_core_info()
    nc, ns = info.num_cores, info.num_subcores
    nw = nc * ns
    rows_per_w = n_rows // nw
    n_chunks = rows_per_w // _CHUNK
    assert rows_per_w * nw == n_rows and n_chunks * _CHUNK == rows_per_w
    assert n_chunks % _NIN == 0 and n_emb % 2 == 0

    idx = x.reshape(n_rows).astype(jnp.int32)
    table_w = emb_table.reshape(n_emb // 2, _WIDE)
    mesh = plsc.VectorSubcoreMesh(core_axis_name="c", subcore_axis_name="s")

    @functools.partial(
        pl.kernel,
        out_type=jax.ShapeDtypeStruct((n_rows, _D), jnp.float32),
        mesh=mesh,
        compiler_params=pltpu.CompilerParams(
            needs_layout_passes=False, use_tc_tiling_on_sc=False),
        scratch_types=[
            pltpu.VMEM((rows_per_w,), jnp.int32),
            pltpu.VMEM((_NIN, _CHUNK), jnp.int32),
            pltpu.VMEM((_NIN, _CHUNK, _WIDE), jnp.float32),
            pltpu.VMEM((_NOUT, _CHUNK, _D), jnp.float32),
            pltpu.VMEM((_D,), jnp.float32),
            pltpu.VMEM((_D,), jnp.float32),
        ]
        + [pltpu.SemaphoreType.DMA] * (_NIN + _NOUT),
    )
    def run(table_hbm, idx_hbm, gamma_hbm, beta_hbm, out_hbm,
            idx_v, wide_v, in_v, out_v, gamma_v, beta_v, *sems):
        wid = lax.axis_index("s") * nc + lax.axis_index("c")
        pltpu.sync_copy(idx_hbm.at[pl.ds(wid * rows_per_w, rows_per_w)], idx_v)
        pltpu.sync_copy(gamma_hbm, gamma_v)
        pltpu.sync_copy(beta_hbm, beta_v)
        lanes = lax.iota(jnp.int32, _LANES)
        sem_in = sems[:_NIN]
        sem_out = sems[_NIN:]

        def out_slice(j):
            return out_hbm.at[pl.ds(wid * rows_per_w + j * _CHUNK, _CHUNK)]

        def start_gather(j, b):
            for k in range(_CHUNK // _LANES):
                iv = idx_v[pl.ds(j * _CHUNK + k * _LANES, _LANES)]
                wide_v.at[b][pl.ds(k * _LANES, _LANES)] = (
                    lax.shift_right_logical(iv, 1))
            pltpu.async_copy(table_hbm.at[wide_v.at[b]], in_v.at[b], sem_in[b])

        for b in range(_NIN):
            start_gather(b, b)

        def compute(j, src, dst):
            poffs, rows = [], []
            for g in range(_NGRP):
                iv = idx_v[pl.ds(j * _CHUNK + g * _LANES, _LANES)]
                poffs.append(lax.shift_left(iv & 1, 6))
                rows.append(g * _LANES + lanes)

            def p1_body(d, carry):
                col = (d + lanes) & 63
                new = []
                for g in range(_NGRP):
                    s1, s2 = carry[2 * g], carry[2 * g + 1]
                    v = plsc.load_gather(src, [rows[g], poffs[g] + col])
                    new.append(s1 + v)
                    new.append(s2 + v * v)
                return tuple(new)

            init = (jnp.zeros((_LANES,), jnp.float32),) * (2 * _NGRP)
            acc = plsc.parallel_loop(0, _D, carry=init, unroll=4)(p1_body)

            means, rs = [], []
            for g in range(_NGRP):
                s1, s2 = acc[2 * g], acc[2 * g + 1]
                mean = s1 * (1.0 / _D)
                var = s2 * (1.0 / _D) - mean * mean
                means.append(mean)
                rs.append(_rsqrt(var + _EPS))

            @plsc.parallel_loop(0, _D, unroll=4)
            def p2_body(d):
                col = (d + lanes) & 63
                gd = plsc.load_gather(gamma_v, [col])
                bd = plsc.load_gather(beta_v, [col])
                for g in range(_NGRP):
                    v = plsc.load_gather(src, [rows[g], poffs[g] + col])
                    o = (v - means[g]) * rs[g] * gd + bd
                    plsc.store_scatter(dst, [rows[g], col], o)

        @pl.loop(0, n_chunks // _NIN)
        def outer(t):
            for b in range(_NIN):
                j = t * _NIN + b
                bo = b % _NOUT
                pltpu.make_async_copy(
                    table_hbm.at[wide_v.at[b]], in_v.at[b], sem_in[b]).wait()

                if b >= _NOUT:
                    pltpu.make_async_copy(
                        out_v.at[bo], out_slice(j), sem_out[bo]).wait()
                else:
                    @pl.when(t > 0)
                    def _():
                        pltpu.make_async_copy(
                            out_v.at[bo], out_slice(j), sem_out[bo]).wait()

                compute(j, in_v.at[b], out_v.at[bo])

                pltpu.async_copy(out_v.at[bo], out_slice(j), sem_out[bo])

                @pl.when(t < n_chunks // _NIN - 1)
                def _():
                    start_gather(j + _NIN, b)

        for b in range(_NOUT):
            pltpu.make_async_copy(
                out_v.at[b], out_slice(n_chunks - _NOUT + b),
                sem_out[b]).wait()

    out = run(table_w, idx, ln_weight, ln_bias)
    return out.reshape(batch, seq, _D)


# speedup vs baseline: 1.0589x; 1.0589x over previous
import functools

import jax
import jax.numpy as jnp
from jax import lax
from jax.experimental import pallas as pl
from jax.experimental.pallas import tpu as pltpu
from jax.experimental.pallas import tpu_sc as plsc

_D = 64
_WIDE = 128
_CHUNK = 256
_SUB = 128
_LANES = 16
_NGRP = _SUB // _LANES
_NIN = 2
_NOUT = 2
_EPS = 1e-5


def _rsqrt(x):
    i = plsc.bitcast(x, jnp.int32)
    i = jnp.int32(0x5F3759DF) - lax.shift_right_logical(i, 1)
    y = plsc.bitcast(i, jnp.float32)
    half = x * 0.5
    for _ in range(3):
        y = y * (1.5 - half * y * y)
    return y


def kernel(x, emb_table, ln_weight, ln_bias):
    batch, seq = x.shape
    n_rows = batch * seq
    n_emb = emb_table.shape[0]
    info = plsc.get_sparse_core_info()
    nc, ns = info.num_cores, info.num_subcores
    nw = nc * ns
    rows_per_w = n_rows // nw
    n_chunks = rows_per_w // _CHUNK
    assert rows_per_w * nw == n_rows and n_chunks * _CHUNK == rows_per_w
    assert n_chunks % _NIN == 0 and n_emb % 2 == 0

    idx = x.reshape(n_rows).astype(jnp.int32)
    table_w = emb_table.reshape(n_emb // 2, _WIDE)
    mesh = plsc.VectorSubcoreMesh(core_axis_name="c", subcore_axis_name="s")

    @functools.partial(
        pl.kernel,
        out_type=jax.ShapeDtypeStruct((n_rows // 2, _WIDE), jnp.float32),
        mesh=mesh,
        compiler_params=pltpu.CompilerParams(
            needs_layout_passes=False, use_tc_tiling_on_sc=True),
        scratch_types=[
            pltpu.VMEM((rows_per_w,), jnp.int32),
        ]
        + [pltpu.VMEM((_CHUNK,), jnp.int32)] * _NIN
        + [pltpu.VMEM((_CHUNK, _WIDE), jnp.float32)] * _NIN
        + [pltpu.VMEM((_CHUNK // 2, _WIDE), jnp.float32)] * _NOUT
        + [
            pltpu.VMEM((_D,), jnp.float32),
            pltpu.VMEM((_D,), jnp.float32),
        ]
        + [pltpu.SemaphoreType.DMA] * (_NIN + _NOUT),
    )
    def run(table_hbm, idx_hbm, gamma_hbm, beta_hbm, out_hbm,
            idx_v, *rest):
        wide_v = rest[:_NIN]
        in_v = rest[_NIN:2 * _NIN]
        out_v = rest[2 * _NIN:2 * _NIN + _NOUT]
        gamma_v, beta_v = rest[2 * _NIN + _NOUT:2 * _NIN + _NOUT + 2]
        sems = rest[2 * _NIN + _NOUT + 2:]
        wid = lax.axis_index("s") * nc + lax.axis_index("c")
        pltpu.sync_copy(
            idx_hbm.at[pl.ds(pl.multiple_of(wid * rows_per_w, 1024),
                             rows_per_w)], idx_v)
        pltpu.sync_copy(gamma_hbm, gamma_v)
        pltpu.sync_copy(beta_hbm, beta_v)
        lanes = lax.iota(jnp.int32, _LANES)
        sem_in = sems[:_NIN]
        sem_out = sems[_NIN:]

        def out_slice(j):
            base = (wid * rows_per_w + j * _CHUNK) // 2
            return out_hbm.at[pl.ds(pl.multiple_of(base, 128), _CHUNK // 2)]

        def start_gather(j, b):
            for k in range(_CHUNK // _LANES):
                iv = idx_v[pl.ds(j * _CHUNK + k * _LANES, _LANES)]
                wide_v[b][pl.ds(k * _LANES, _LANES)] = (
                    lax.shift_right_logical(iv, 1))
            pltpu.async_copy(table_hbm.at[wide_v[b]], in_v[b], sem_in[b])

        for b in range(_NIN):
            start_gather(b, b)

        def compute_sub(j, src, dst, base):
            poffs, rows, wrows, wpre = [], [], [], []
            for g in range(_NGRP):
                iv = idx_v[pl.ds(j * _CHUNK + base + g * _LANES, _LANES)]
                poffs.append(lax.shift_left(iv & 1, 6))
                r = base + g * _LANES + lanes
                rows.append(r)
                wrows.append(lax.shift_right_logical(r, 1))
                wpre.append(lax.shift_left(r & 1, 6))

            def p1_body(d, carry):
                col = (d + lanes) & 63
                new = []
                for g in range(_NGRP):
                    s1, s2 = carry[2 * g], carry[2 * g + 1]
                    v = plsc.load_gather(src, [rows[g], poffs[g] + col])
                    new.append(s1 + v)
                    new.append(s2 + v * v)
                return tuple(new)

            init = (jnp.zeros((_LANES,), jnp.float32),) * (2 * _NGRP)
            acc = plsc.parallel_loop(0, _D, carry=init, unroll=4)(p1_body)

            means, rs = [], []
            for g in range(_NGRP):
                s1, s2 = acc[2 * g], acc[2 * g + 1]
                mean = s1 * (1.0 / _D)
                var = s2 * (1.0 / _D) - mean * mean
                means.append(mean)
                rs.append(_rsqrt(var + _EPS))

            @plsc.parallel_loop(0, _D, unroll=4)
            def p2_body(d):
                col = (d + lanes) & 63
                gd = plsc.load_gather(gamma_v, [col])
                bd = plsc.load_gather(beta_v, [col])
                for g in range(_NGRP):
                    v = plsc.load_gather(src, [rows[g], poffs[g] + col])
                    o = (v - means[g]) * rs[g] * gd + bd
                    plsc.store_scatter(dst, [wrows[g], wpre[g] + col], o)

        @pl.loop(0, n_chunks // _NIN)
        def outer(t):
            for b in range(_NIN):
                j = t * _NIN + b
                bo = b % _NOUT
                pltpu.make_async_copy(
                    table_hbm.at[wide_v[b]], in_v[b], sem_in[b]).wait()

                @pl.when(t > 0)
                def _():
                    pltpu.make_async_copy(
                        out_v[bo], out_slice(j), sem_out[bo]).wait()

                for sb in range(_CHUNK // _SUB):
                    compute_sub(j, in_v[b], out_v[bo], sb * _SUB)

                pltpu.async_copy(out_v[bo], out_slice(j), sem_out[bo])

                @pl.when(t < n_chunks // _NIN - 1)
                def _():
                    start_gather(j + _NIN, b)

        for b in range(_NOUT):
            pltpu.make_async_copy(
                out_v[b], out_slice(n_chunks - _NOUT + b),
                sem_out[b]).wait()

    out = run(table_w, idx, ln_weight, ln_bias)
    return out.reshape(batch, seq, _D)


# speedup vs baseline: 1.6334x; 1.5425x over previous
import functools

import jax
import jax.numpy as jnp
from jax import lax
from jax.experimental import pallas as pl
from jax.experimental.pallas import tpu as pltpu
from jax.experimental.pallas import tpu_sc as plsc

_D = 64
_CHUNK = 256
_SUB = 128
_LANES = 16
_NGRP = _SUB // _LANES
_NIN = 4
_NOUT = 2
_EPS = 1e-5


def _rsqrt(x):
    i = plsc.bitcast(x, jnp.int32)
    i = jnp.int32(0x5F3759DF) - lax.shift_right_logical(i, 1)
    y = plsc.bitcast(i, jnp.float32)
    half = x * 0.5
    for _ in range(3):
        y = y * (1.5 - half * y * y)
    return y


def kernel(x, emb_table, ln_weight, ln_bias):
    batch, seq = x.shape
    n_rows = batch * seq
    info = plsc.get_sparse_core_info()
    nc, ns = info.num_cores, info.num_subcores
    nw = nc * ns
    rows_per_w = n_rows // nw
    n_chunks = rows_per_w // _CHUNK
    nbt = batch // 128
    nbh = nbt // 2
    assert rows_per_w * nw == n_rows and n_chunks * _CHUNK == rows_per_w
    assert n_chunks % _NIN == 0

    idx = x.T.reshape(n_rows).astype(jnp.int32)
    mesh = plsc.VectorSubcoreMesh(core_axis_name="c", subcore_axis_name="s")

    @functools.partial(
        pl.kernel,
        out_type=jax.ShapeDtypeStruct((seq, _D // 8, nbt, 8, 128),
                                      jnp.float32),
        mesh=mesh,
        compiler_params=pltpu.CompilerParams(
            needs_layout_passes=False, use_tc_tiling_on_sc=False),
        scratch_types=[
            pltpu.VMEM((rows_per_w,), jnp.int32),
            pltpu.VMEM((_NIN, _CHUNK, _D), jnp.float32),
            pltpu.VMEM((_NOUT, _D // 8, 2, 8, 128), jnp.float32),
            pltpu.VMEM((_D,), jnp.float32),
            pltpu.VMEM((_D,), jnp.float32),
        ]
        + [pltpu.SemaphoreType.DMA] * (_NIN + _NOUT),
    )
    def run(table_hbm, idx_hbm, gamma_hbm, beta_hbm, out_hbm,
            idx_v, in_v, out_v, gamma_v, beta_v, *sems):
        wid = lax.axis_index("s") * nc + lax.axis_index("c")
        pltpu.sync_copy(idx_hbm.at[pl.ds(wid * rows_per_w, rows_per_w)], idx_v)
        pltpu.sync_copy(gamma_hbm, gamma_v)
        pltpu.sync_copy(beta_hbm, beta_v)
        lanes = lax.iota(jnp.int32, _LANES)
        sem_in = sems[:_NIN]
        sem_out = sems[_NIN:]

        def idx_slice(j):
            return idx_v.at[pl.ds(j * _CHUNK, _CHUNK)]

        def out_slice(j):
            c = wid * n_chunks + j
            sv = c // nbh
            bh = c % nbh
            return out_hbm.at[sv, :, pl.ds(bh * 2, 2)]

        for b in range(_NIN):
            pltpu.async_copy(table_hbm.at[idx_slice(b)], in_v.at[b], sem_in[b])

        def compute_sub(src, dst, base):
            rb = base // 128
            bc0 = base % 128
            rows, bcs = [], []
            for g in range(_NGRP):
                rows.append(base + g * _LANES + lanes)
                bcs.append(bc0 + g * _LANES + lanes)
            rbv = jnp.full((_LANES,), rb, jnp.int32)

            def p1_body(d, carry):
                col = (d + lanes) & 63
                new = []
                for g in range(_NGRP):
                    s1, s2 = carry[2 * g], carry[2 * g + 1]
                    v = plsc.load_gather(src, [rows[g], col])
                    new.append(s1 + v)
                    new.append(s2 + v * v)
                return tuple(new)

            init = (jnp.zeros((_LANES,), jnp.float32),) * (2 * _NGRP)
            acc = plsc.parallel_loop(0, _D, carry=init, unroll=4)(p1_body)

            means, rs = [], []
            for g in range(_NGRP):
                s1, s2 = acc[2 * g], acc[2 * g + 1]
                mean = s1 * (1.0 / _D)
                var = s2 * (1.0 / _D) - mean * mean
                means.append(mean)
                rs.append(_rsqrt(var + _EPS))

            @plsc.parallel_loop(0, _D, unroll=4)
            def p2_body(d):
                col = (d + lanes) & 63
                dhi = lax.shift_right_logical(col, 3)
                dlo = col & 7
                gd = plsc.load_gather(gamma_v, [col])
                bd = plsc.load_gather(beta_v, [col])
                for g in range(_NGRP):
                    v = plsc.load_gather(src, [rows[g], col])
                    o = (v - means[g]) * rs[g] * gd + bd
                    plsc.store_scatter(dst, [dhi, rbv, dlo, bcs[g]], o)

        @pl.loop(0, n_chunks // _NIN)
        def outer(t):
            for b in range(_NIN):
                j = t * _NIN + b
                bo = b % _NOUT
                pltpu.make_async_copy(
                    table_hbm.at[idx_slice(j)], in_v.at[b], sem_in[b]).wait()

                if b >= _NOUT:
                    pltpu.make_async_copy(
                        out_v.at[bo], out_slice(j), sem_out[bo]).wait()
                else:
                    @pl.when(t > 0)
                    def _():
                        pltpu.make_async_copy(
                            out_v.at[bo], out_slice(j), sem_out[bo]).wait()

                for sb in range(_CHUNK // _SUB):
                    compute_sub(in_v.at[b], out_v.at[bo], sb * _SUB)

                pltpu.async_copy(out_v.at[bo], out_slice(j), sem_out[bo])

                @pl.when(t < n_chunks // _NIN - 1)
                def _():
                    pltpu.async_copy(
                        table_hbm.at[idx_slice(j + _NIN)], in_v.at[b],
                        sem_in[b])

        for b in range(_NOUT):
            pltpu.make_async_copy(
                out_v.at[b], out_slice(n_chunks - _NOUT + b),
                sem_out[b]).wait()

    out = run(emb_table, idx, ln_weight, ln_bias)
    return out.transpose(2, 4, 0, 1, 3).reshape(batch, seq, _D)
